# baseline (speedup 1.0000x reference)
import functools

import jax
import jax.numpy as jnp
from jax import lax
from jax.experimental import pallas as pl
from jax.experimental.pallas import tpu as pltpu
from jax.experimental.pallas import tpu_sc as plsc

_B, _C, _R = 2, 96, 48
_S = 32 * 64 * 64
_D = 4096
_RPS = _S // _D
_NROW_OUT = _B * _R * _RPS
_CHUNK = 8


def _mean_kernel(x_ref, mean_ref):
    blk = x_ref[...]
    mean_ref[...] = (jnp.sum(blk, axis=2) * (1.0 / _S)).reshape(1, 1, 8)


def _mlp_rank_kernel(y_ref, w1_ref, b1_ref, w2_ref, b2_ref, idx_ref):
    y = y_ref[...]
    z1 = lax.dot_general(y, w1_ref[...], (((1,), (1,)), ((), ())),
                         preferred_element_type=jnp.float32)
    z1 = z1 + b1_ref[...]
    z1 = jnp.where(z1 >= 0, z1, 0.01 * z1)
    z2 = lax.dot_general(z1, w2_ref[...], (((1,), (1,)), ((), ())),
                         preferred_element_type=jnp.float32)
    z2 = z2 + b2_ref[...]
    s = 1.0 / (1.0 + jnp.exp(-z2))

    for b in range(_B):
        vrow = s[b:b + 1, :]
        rmat = jnp.broadcast_to(vrow, (_C, _C))
        cmat = rmat.T
        irow = lax.broadcasted_iota(jnp.int32, (_C, _C), 0)
        jcol = lax.broadcasted_iota(jnp.int32, (_C, _C), 1)
        before = (cmat > rmat) | ((cmat == rmat) & (irow < jcol))
        rank = jnp.sum(before.astype(jnp.int32), axis=0, keepdims=True)
        pmat = lax.broadcasted_iota(jnp.int32, (_R, _C), 0)
        rkmat = jnp.broadcast_to(rank, (_R, _C))
        cio = lax.broadcasted_iota(jnp.int32, (_R, _C), 1)
        src = jnp.sum(jnp.where(rkmat == pmat, cio, 0), axis=1,
                      keepdims=True)
        kio = lax.broadcasted_iota(jnp.int32, (_R, _RPS), 1)
        idx_ref[pl.ds(b * _R, _R), :] = (src + b * _C) * _RPS + kio


def _tc_stage(x3, w1, b1, w2, b2):
    nblk = _B * _C // 8
    means = pl.pallas_call(
        _mean_kernel,
        grid=(_B, _C // 8),
        in_specs=[pl.BlockSpec((1, 8, _S), lambda b, j: (b, j, 0))],
        out_specs=pl.BlockSpec((1, 1, 8), lambda b, j: (b * (_C // 8) + j, 0, 0)),
        out_shape=jax.ShapeDtypeStruct((nblk, 1, 8), jnp.float32),
    )(x3)
    return pl.pallas_call(
        _mlp_rank_kernel,
        in_specs=[
            pl.BlockSpec((_B, _C), lambda: (0, 0)),
            pl.BlockSpec((_C, _C), lambda: (0, 0)),
            pl.BlockSpec((1, _C), lambda: (0, 0)),
            pl.BlockSpec((_C, _C), lambda: (0, 0)),
            pl.BlockSpec((1, _C), lambda: (0, 0)),
        ],
        out_specs=pl.BlockSpec((_B * _R, _RPS), lambda: (0, 0)),
        out_shape=jax.ShapeDtypeStruct((_B * _R, _RPS), jnp.int32),
    )(means.reshape(_B, _C), w1, b1.reshape(1, _C), w2, b2.reshape(1, _C))


def _sc_gather(xrows, idx2d):
    info = plsc.get_sparse_core_info()
    nw = info.num_cores * info.num_subcores
    rows_per_w = _NROW_OUT // nw
    nchunks = rows_per_w // _CHUNK
    mesh = plsc.VectorSubcoreMesh(core_axis_name="c", subcore_axis_name="s")

    @functools.partial(
        pl.kernel, mesh=mesh,
        out_type=jax.ShapeDtypeStruct((_NROW_OUT, _D), jnp.float32),
        scratch_types=[
            pltpu.VMEM((rows_per_w,), jnp.int32),
            pltpu.VMEM((_CHUNK, _D), jnp.float32),
            pltpu.SemaphoreType.DMA,
        ],
    )
    def gk(x_hbm, idx_hbm, out_hbm, idx_v, rows_v, sem):
        wid = lax.axis_index("s") * info.num_cores + lax.axis_index("c")
        base = wid * rows_per_w
        pltpu.sync_copy(idx_hbm.at[pl.ds(base, rows_per_w)], idx_v)
        for c in range(nchunks):
            pltpu.async_copy(x_hbm.at[idx_v.at[pl.ds(c * _CHUNK, _CHUNK)]],
                             rows_v, sem).wait()
            pltpu.sync_copy(rows_v, out_hbm.at[pl.ds(base + c * _CHUNK,
                                                     _CHUNK)])

    return gk(xrows, idx2d)


def kernel(x, w1, b1, w2, b2):
    b, c, d, h, w = x.shape
    x3 = x.reshape(_B, _C, _S)
    idx = _tc_stage(x3, w1, b1, w2, b2)
    xrows = x.reshape(_B * _C * _RPS, _D)
    out = _sc_gather(xrows, idx.reshape(-1))
    return out.reshape(b, _R, d, h, w)


# speedup vs baseline: 2.2141x; 2.2141x over previous
import jax
import jax.numpy as jnp
from jax import lax
from jax.experimental import pallas as pl
from jax.experimental.pallas import tpu as pltpu

_B, _C, _R = 2, 96, 48
_S = 32 * 64 * 64
_CHA = 8192
_NCH = _S // _CHA
_CHB = 4096


def _mean_kernel(x_ref, part_ref):
    blk = x_ref[...]
    part_ref[...] = jnp.sum(blk, axis=1, keepdims=True).reshape(1, 1, 1, _C)


def _mlp_rank_onehot_kernel(part_ref, w1_ref, b1_ref, w2_ref, b2_ref, p_ref):
    part = part_ref[...].reshape(_B, _NCH, _C)
    y = jnp.sum(part, axis=1) * (1.0 / _S)
    z1 = lax.dot_general(y, w1_ref[...], (((1,), (1,)), ((), ())),
                         preferred_element_type=jnp.float32)
    z1 = z1 + b1_ref[...]
    z1 = jnp.where(z1 >= 0, z1, 0.01 * z1)
    z2 = lax.dot_general(z1, w2_ref[...], (((1,), (1,)), ((), ())),
                         preferred_element_type=jnp.float32)
    z2 = z2 + b2_ref[...]
    s = 1.0 / (1.0 + jnp.exp(-z2))

    for b in range(_B):
        vrow = s[b:b + 1, :]
        rmat = jnp.broadcast_to(vrow, (_C, _C))
        cmat = rmat.T
        irow = lax.broadcasted_iota(jnp.int32, (_C, _C), 0)
        jcol = lax.broadcasted_iota(jnp.int32, (_C, _C), 1)
        beforeT = (rmat > cmat) | ((rmat == cmat) & (jcol < irow))
        rank = jnp.sum(beforeT.astype(jnp.int32), axis=1,
                       keepdims=True)
        pio = lax.broadcasted_iota(jnp.int32, (_C, _R), 1)
        onehot = (jnp.broadcast_to(rank, (_C, _R)) == pio)
        p_ref[b] = onehot.astype(jnp.float32)


def _gather_mm_kernel(x_ref, p_ref, o_ref):
    x2 = x_ref[...].reshape(_CHB, _C)
    p2 = p_ref[...].reshape(_C, _R)
    o = lax.dot_general(x2, p2, (((1,), (0,)), ((), ())),
                        preferred_element_type=jnp.float32)
    o_ref[...] = o.reshape(1, _CHB, _R)


def kernel(x, w1, b1, w2, b2):
    b, c, d, h, w = x.shape
    xt = jnp.transpose(x, (0, 2, 3, 4, 1))
    xv = xt.reshape(_B, _S, _C)

    part = pl.pallas_call(
        _mean_kernel,
        grid=(_B, _NCH),
        in_specs=[pl.BlockSpec((1, _CHA, _C), lambda i, j: (i, j, 0))],
        out_specs=pl.BlockSpec((1, 1, 1, _C), lambda i, j: (i, j, 0, 0)),
        out_shape=jax.ShapeDtypeStruct((_B, _NCH, 1, _C), jnp.float32),
    )(xv)

    onehot = pl.pallas_call(
        _mlp_rank_onehot_kernel,
        in_specs=[
            pl.BlockSpec((_B, _NCH, 1, _C), lambda: (0, 0, 0, 0)),
            pl.BlockSpec((_C, _C), lambda: (0, 0)),
            pl.BlockSpec((1, _C), lambda: (0, 0)),
            pl.BlockSpec((_C, _C), lambda: (0, 0)),
            pl.BlockSpec((1, _C), lambda: (0, 0)),
        ],
        out_specs=pl.BlockSpec((_B, _C, _R), lambda: (0, 0, 0)),
        out_shape=jax.ShapeDtypeStruct((_B, _C, _R), jnp.float32),
    )(part, w1, b1.reshape(1, _C), w2, b2.reshape(1, _C))

    out_t = pl.pallas_call(
        _gather_mm_kernel,
        grid=(_B, _S // _CHB),
        in_specs=[
            pl.BlockSpec((1, _CHB, _C), lambda i, j: (i, j, 0)),
            pl.BlockSpec((1, _C, _R), lambda i, j: (i, 0, 0)),
        ],
        out_specs=pl.BlockSpec((1, _CHB, _R), lambda i, j: (i, j, 0)),
        out_shape=jax.ShapeDtypeStruct((_B, _S, _R), jnp.float32),
    )(xv, onehot)

    return out_t.reshape(b, d, h, w, _R).transpose(0, 4, 1, 2, 3)
